# Initial kernel scaffold; baseline (speedup 1.0000x reference)
import jax
import jax.numpy as jnp
from jax.experimental import pallas as pl


def kernel(inp_features, inp_positions, out_positions, voxel_size, kernel, bias):
    raise NotImplementedError("write your pallas kernel here")


# speedup vs baseline: 2.0058x; 2.0058x over previous
import functools

import jax
import jax.numpy as jnp
from jax import lax
from jax.experimental import pallas as pl
from jax.experimental.pallas import tpu as pltpu
from jax.experimental.pallas import tpu_sc as plsc

G = 40
GP = G + 2
NCELL = GP * GP * GP
HALF = 37120
RPAD = 2 * HALF
DUMMY = HALF
SH_ROWS = HALF + 8
CIN = 32
COUT = 32
NP = 10240
PPT = 640
GPT = 320
CH = 32
NTAP = 27

OFFS = [(dx * GP + dy) * GP + dz
        for dx in (-1, 0, 1) for dy in (-1, 0, 1) for dz in (-1, 0, 1)]

_MESH = plsc.VectorSubcoreMesh(core_axis_name="c", subcore_axis_name="s")
_SC_PARAMS = pltpu.CompilerParams(use_tc_tiling_on_sc=False)


@functools.partial(
    pl.kernel,
    mesh=_MESH,
    out_type=jax.ShapeDtypeStruct((RPAD, CIN), jnp.float32),
    scratch_types=[
        pltpu.VMEM((PPT,), jnp.int32),
        pltpu.VMEM((5, 128), jnp.int32),
        pltpu.VMEM((128, CIN), jnp.float32),
        pltpu.VMEM_SHARED((SH_ROWS, CIN), jnp.float32),
    ],
    compiler_params=_SC_PARAMS,
)
def _scatter_grid(lin_hbm, feat_hbm, zeros_hbm, grid_hbm,
                  linbuf, idxbuf, featbuf, shared):
    c = lax.axis_index("c")
    s = lax.axis_index("s")
    pltpu.sync_copy(zeros_hbm.at[pl.ds(s * 2320, 2320)],
                    shared.at[pl.ds(s * 2320, 2320)])
    pltpu.sync_copy(lin_hbm.at[pl.ds(s * PPT, PPT)], linbuf)
    base = c * HALF
    basev = jnp.broadcast_to(base, (16,))
    for k in range(5):
        for j in range(8):
            lv = linbuf[pl.ds(k * 128 + j * 16, 16)]
            loc = lv - basev
            ok = (loc >= 0) & (loc < HALF)
            idxbuf[k, pl.ds(j * 16, 16)] = jnp.where(ok, loc, DUMMY)
    plsc.subcore_barrier()
    for k in range(5):
        pltpu.sync_copy(feat_hbm.at[pl.ds(s * PPT + k * 128, 128)], featbuf)
        pltpu.sync_copy(featbuf, shared.at[idxbuf.at[k]], add=True)
    plsc.subcore_barrier()
    pltpu.sync_copy(shared.at[pl.ds(s * 2320, 2320)],
                    grid_hbm.at[pl.ds(base + s * 2320, 2320)])


@functools.partial(
    pl.kernel,
    mesh=_MESH,
    out_type=jax.ShapeDtypeStruct((NTAP, NP, CIN), jnp.float32),
    scratch_types=[
        pltpu.VMEM((GPT,), jnp.int32),
        pltpu.VMEM((NTAP, CH), jnp.int32),
        pltpu.VMEM((NTAP, CH, CIN), jnp.float32),
        pltpu.SemaphoreType.DMA,
        pltpu.SemaphoreType.DMA,
    ],
    compiler_params=_SC_PARAMS,
)
def _gather_cols(grid_hbm, lin_hbm, col_hbm, linbuf, idxbuf, gbuf, gsem, wsem):
    c = lax.axis_index("c")
    s = lax.axis_index("s")
    w = s * 2 + c
    base = w * GPT
    pltpu.sync_copy(lin_hbm.at[pl.ds(base, GPT)], linbuf)
    for ch in range(GPT // CH):
        for v in range(CH // 16):
            bv = linbuf[pl.ds(ch * CH + v * 16, 16)]
            for t in range(NTAP):
                idxbuf[t, pl.ds(v * 16, 16)] = bv + OFFS[t]
        descs = [pltpu.async_copy(grid_hbm.at[idxbuf.at[t]], gbuf.at[t], gsem)
                 for t in range(NTAP)]
        for d in descs:
            d.wait()
        wdescs = [pltpu.async_copy(
            gbuf.at[t],
            col_hbm.at[t, pl.ds(base + ch * CH, CH)],
            wsem) for t in range(NTAP)]
        for d in wdescs:
            d.wait()


def _mm_body(col_ref, w_ref, b_ref, o_ref):
    acc = jnp.broadcast_to(b_ref[...], (512, COUT))
    for t in range(NTAP):
        acc = acc + jnp.dot(col_ref[t], w_ref[t],
                            preferred_element_type=jnp.float32)
    o_ref[...] = acc


def kernel(inp_features, inp_positions, out_positions, voxel_size, kernel, bias):
    n_in = inp_features.shape[0]
    n_out = out_positions.shape[0]
    v = jnp.float32(voxel_size)
    ic = jnp.round(inp_positions / v).astype(jnp.int32) + 1
    oc = jnp.round(out_positions / v).astype(jnp.int32) + 1
    lin_in = (ic[:, 0] * GP + ic[:, 1]) * GP + ic[:, 2]
    lin_out = (oc[:, 0] * GP + oc[:, 1]) * GP + oc[:, 2]

    lin_in_p = jnp.full((NP,), -1, jnp.int32).at[:n_in].set(lin_in)
    feats_p = jnp.zeros((NP, CIN), jnp.float32).at[:n_in].set(inp_features)
    lin_out_p = jnp.full((NP,), GP * GP + GP + 1, jnp.int32).at[:n_out].set(lin_out)
    zeros_sh = jnp.zeros((HALF, CIN), jnp.float32)

    grid = _scatter_grid(lin_in_p, feats_p, zeros_sh)
    col = _gather_cols(grid, lin_out_p)

    w2 = kernel.reshape(NTAP, CIN, COUT)
    b2 = bias.reshape(1, COUT)
    out = pl.pallas_call(
        _mm_body,
        grid=(NP // 512,),
        in_specs=[pl.BlockSpec((NTAP, 512, CIN), lambda i: (0, i, 0)),
                  pl.BlockSpec((NTAP, CIN, COUT), lambda i: (0, 0, 0)),
                  pl.BlockSpec((1, COUT), lambda i: (0, 0))],
        out_specs=pl.BlockSpec((512, COUT), lambda i: (i, 0)),
        out_shape=jax.ShapeDtypeStruct((NP, COUT), jnp.float32),
    )(col, w2, b2)
    return out[:n_out]


# speedup vs baseline: 3.7923x; 1.8907x over previous
import functools

import jax
import jax.numpy as jnp
from jax import lax
from jax.experimental import pallas as pl
from jax.experimental.pallas import tpu as pltpu
from jax.experimental.pallas import tpu_sc as plsc

G = 40
GP = G + 2
NCELL = GP * GP * GP
HALF = 37120
RPAD = 2 * HALF
DUMMY = HALF
SH_ROWS = HALF + 8
CIN = 32
COUT = 32
NP = 10240
PPT = 640
GPT = 320
CH = 32
NTAP = 27

OFFS = [(dx * GP + dy) * GP + dz
        for dx in (-1, 0, 1) for dy in (-1, 0, 1) for dz in (-1, 0, 1)]
OFFS28 = OFFS + [0]
NQUAD = 7

_MESH = plsc.VectorSubcoreMesh(core_axis_name="c", subcore_axis_name="s")
_SC_PARAMS = pltpu.CompilerParams(use_tc_tiling_on_sc=False)


@functools.partial(
    pl.kernel,
    mesh=_MESH,
    out_type=jax.ShapeDtypeStruct((RPAD, CIN), jnp.float32),
    scratch_types=[
        pltpu.VMEM((PPT,), jnp.int32),
        pltpu.VMEM((5, 128), jnp.int32),
        pltpu.VMEM((128, CIN), jnp.float32),
        pltpu.VMEM_SHARED((SH_ROWS, CIN), jnp.float32),
    ],
    compiler_params=_SC_PARAMS,
)
def _scatter_grid(lin_hbm, feat_hbm, zeros_hbm, grid_hbm,
                  linbuf, idxbuf, featbuf, shared):
    c = lax.axis_index("c")
    s = lax.axis_index("s")
    pltpu.sync_copy(zeros_hbm.at[pl.ds(s * 2320, 2320)],
                    shared.at[pl.ds(s * 2320, 2320)])
    pltpu.sync_copy(lin_hbm.at[pl.ds(s * PPT, PPT)], linbuf)
    base = c * HALF
    basev = jnp.broadcast_to(base, (16,))
    for k in range(5):
        for j in range(8):
            lv = linbuf[pl.ds(k * 128 + j * 16, 16)]
            loc = lv - basev
            ok = (loc >= 0) & (loc < HALF)
            idxbuf[k, pl.ds(j * 16, 16)] = jnp.where(ok, loc, DUMMY)
    plsc.subcore_barrier()
    for k in range(5):
        pltpu.sync_copy(feat_hbm.at[pl.ds(s * PPT + k * 128, 128)], featbuf)
        pltpu.sync_copy(featbuf, shared.at[idxbuf.at[k]], add=True)
    plsc.subcore_barrier()
    pltpu.sync_copy(shared.at[pl.ds(s * 2320, 2320)],
                    grid_hbm.at[pl.ds(base + s * 2320, 2320)])


NCHUNK = NP // CH
CPW = GPT // CH


@functools.partial(
    pl.kernel,
    mesh=_MESH,
    out_type=jax.ShapeDtypeStruct((NCHUNK, NQUAD, CH, 128), jnp.float32),
    scratch_types=[
        pltpu.VMEM((GPT,), jnp.int32),
        pltpu.VMEM((4 * NQUAD, CH), jnp.int32),
        pltpu.VMEM((4 * NQUAD, CH, CIN), jnp.float32),
        pltpu.SemaphoreType.DMA,
        pltpu.SemaphoreType.DMA,
    ],
    compiler_params=_SC_PARAMS,
)
def _gather_cols(grid_hbm, lin_hbm, col_hbm, linbuf, idxbuf, gbuf, gsem, wsem):
    c = lax.axis_index("c")
    s = lax.axis_index("s")
    w = s * 2 + c
    base = w * GPT
    pltpu.sync_copy(lin_hbm.at[pl.ds(base, GPT)], linbuf)
    for ch in range(CPW):
        cid = w * CPW + ch
        for v in range(CH // 16):
            bv = linbuf[pl.ds(ch * CH + v * 16, 16)]
            for t in range(4 * NQUAD):
                idxbuf[t, pl.ds(v * 16, 16)] = bv + OFFS28[t]
        descs = [pltpu.async_copy(grid_hbm.at[idxbuf.at[t]], gbuf.at[t], gsem)
                 for t in range(4 * NQUAD)]
        for d in descs:
            d.wait()
        wdescs = [pltpu.async_copy(
            gbuf.at[t],
            col_hbm.at[cid, t // 4, :, pl.ds((t % 4) * CIN, CIN)],
            wsem) for t in range(4 * NQUAD)]
        for d in wdescs:
            d.wait()


_MMB = 40


def _mm_body(col_ref, w_ref, b_ref, o_ref):
    acc = jnp.broadcast_to(b_ref[...], (_MMB * CH, COUT))
    for g in range(NQUAD):
        x = col_ref[:, g].reshape(_MMB * CH, 128)
        acc = acc + jnp.dot(x, w_ref[g], preferred_element_type=jnp.float32)
    o_ref[...] = acc


def kernel(inp_features, inp_positions, out_positions, voxel_size, kernel, bias):
    n_in = inp_features.shape[0]
    n_out = out_positions.shape[0]
    v = jnp.float32(voxel_size)
    ic = jnp.round(inp_positions / v).astype(jnp.int32) + 1
    oc = jnp.round(out_positions / v).astype(jnp.int32) + 1
    lin_in = (ic[:, 0] * GP + ic[:, 1]) * GP + ic[:, 2]
    lin_out = (oc[:, 0] * GP + oc[:, 1]) * GP + oc[:, 2]

    lin_in_p = jnp.full((NP,), -1, jnp.int32).at[:n_in].set(lin_in)
    feats_p = jnp.zeros((NP, CIN), jnp.float32).at[:n_in].set(inp_features)
    lin_out_p = jnp.full((NP,), GP * GP + GP + 1, jnp.int32).at[:n_out].set(lin_out)
    zeros_sh = jnp.zeros((HALF, CIN), jnp.float32)

    grid = _scatter_grid(lin_in_p, feats_p, zeros_sh)
    col = _gather_cols(grid, lin_out_p)

    w28 = jnp.concatenate(
        [kernel.reshape(NTAP, CIN, COUT),
         jnp.zeros((1, CIN, COUT), jnp.float32)], axis=0)
    wq = w28.reshape(NQUAD, 4 * CIN, COUT)
    b2 = bias.reshape(1, COUT)
    out = pl.pallas_call(
        _mm_body,
        grid=(NCHUNK // _MMB,),
        in_specs=[pl.BlockSpec((_MMB, NQUAD, CH, 128), lambda i: (i, 0, 0, 0)),
                  pl.BlockSpec((NQUAD, 4 * CIN, COUT), lambda i: (0, 0, 0)),
                  pl.BlockSpec((1, COUT), lambda i: (0, 0))],
        out_specs=pl.BlockSpec((_MMB * CH, COUT), lambda i: (i, 0)),
        out_shape=jax.ShapeDtypeStruct((NP, COUT), jnp.float32),
    )(col, wq, b2)
    return out[:n_out]


# speedup vs baseline: 3.9272x; 1.0356x over previous
import functools

import jax
import jax.numpy as jnp
from jax import lax
from jax.experimental import pallas as pl
from jax.experimental.pallas import tpu as pltpu
from jax.experimental.pallas import tpu_sc as plsc

G = 40
GP = G + 2
NCELL = GP * GP * GP
HALF = 37120
RPAD = 2 * HALF
DUMMY = HALF
SH_ROWS = HALF + 8
CIN = 32
COUT = 32
NP = 10240
PPT = 640
GPT = 320
CH = 32
NTAP = 27

OFFS = [(dx * GP + dy) * GP + dz
        for dx in (-1, 0, 1) for dy in (-1, 0, 1) for dz in (-1, 0, 1)]
OFFS28 = OFFS + [0]
NQUAD = 7

_MESH = plsc.VectorSubcoreMesh(core_axis_name="c", subcore_axis_name="s")
_SC_PARAMS = pltpu.CompilerParams(use_tc_tiling_on_sc=False)


@functools.partial(
    pl.kernel,
    mesh=_MESH,
    out_type=jax.ShapeDtypeStruct((RPAD, CIN), jnp.float32),
    scratch_types=[
        pltpu.VMEM((PPT,), jnp.int32),
        pltpu.VMEM((5, 128), jnp.int32),
        pltpu.VMEM((2, 128, CIN), jnp.float32),
        pltpu.VMEM_SHARED((SH_ROWS, CIN), jnp.float32),
        pltpu.SemaphoreType.DMA,
        pltpu.SemaphoreType.DMA,
    ],
    compiler_params=_SC_PARAMS,
)
def _scatter_grid(lin_hbm, feat_hbm, zeros_hbm, grid_hbm,
                  linbuf, idxbuf, featbuf, shared, zsem, fsem):
    c = lax.axis_index("c")
    s = lax.axis_index("s")
    zd = pltpu.async_copy(zeros_hbm.at[pl.ds(s * 2320, 2320)],
                          shared.at[pl.ds(s * 2320, 2320)], zsem)
    ld = pltpu.async_copy(lin_hbm.at[pl.ds(s * PPT, PPT)], linbuf, fsem)
    f0 = pltpu.async_copy(feat_hbm.at[pl.ds(s * PPT, 128)],
                          featbuf.at[0], fsem)
    ld.wait()
    base = c * HALF
    basev = jnp.broadcast_to(base, (16,))
    for k in range(5):
        for j in range(8):
            lv = linbuf[pl.ds(k * 128 + j * 16, 16)]
            loc = lv - basev
            ok = (loc >= 0) & (loc < HALF)
            idxbuf[k, pl.ds(j * 16, 16)] = jnp.where(ok, loc, DUMMY)
    zd.wait()
    plsc.subcore_barrier()
    fprev = f0
    for k in range(5):
        fprev.wait()
        if k + 1 < 5:
            fprev = pltpu.async_copy(
                feat_hbm.at[pl.ds(s * PPT + (k + 1) * 128, 128)],
                featbuf.at[(k + 1) % 2], fsem)
        pltpu.sync_copy(featbuf.at[k % 2], shared.at[idxbuf.at[k]], add=True)
    plsc.subcore_barrier()
    pltpu.sync_copy(shared.at[pl.ds(s * 2320, 2320)],
                    grid_hbm.at[pl.ds(base + s * 2320, 2320)])


NCHUNK = NP // CH
CPW = GPT // CH


@functools.partial(
    pl.kernel,
    mesh=_MESH,
    out_type=jax.ShapeDtypeStruct((NCHUNK, NQUAD, CH, 128), jnp.float32),
    scratch_types=[
        pltpu.VMEM((GPT,), jnp.int32),
        pltpu.VMEM((2, 4 * NQUAD, CH), jnp.int32),
        pltpu.VMEM((4 * NQUAD, CH, CIN), jnp.float32),
        pltpu.SemaphoreType.DMA,
        pltpu.SemaphoreType.DMA,
        pltpu.SemaphoreType.DMA,
    ],
    compiler_params=_SC_PARAMS,
)
def _gather_cols(grid_hbm, lin_hbm, col_hbm, linbuf, idxbuf, gbuf,
                 gsemA, gsemB, wsem):
    c = lax.axis_index("c")
    s = lax.axis_index("s")
    w = s * 2 + c
    base = w * GPT
    NT = 4 * NQUAD
    HT = NT // 2
    pltpu.sync_copy(lin_hbm.at[pl.ds(base, GPT)], linbuf)

    def build_idx(ch):
        ib = idxbuf.at[ch % 2]
        for v in range(CH // 16):
            bv = linbuf[pl.ds(ch * CH + v * 16, 16)]
            for t in range(NT):
                ib[t, pl.ds(v * 16, 16)] = bv + OFFS28[t]

    def fire_gathers(ch, lo, hi, sem):
        ib = idxbuf.at[ch % 2]
        return [pltpu.async_copy(grid_hbm.at[ib.at[t]], gbuf.at[t], sem)
                for t in range(lo, hi)]

    def fire_writes(cid, lo, hi):
        return [pltpu.async_copy(
            gbuf.at[t],
            col_hbm.at[cid, t // 4, :, pl.ds((t % 4) * CIN, CIN)],
            wsem) for t in range(lo, hi)]

    build_idx(0)
    for ch in range(CPW):
        cid = w * CPW + ch
        ga = fire_gathers(ch, 0, HT, gsemA)
        gb = fire_gathers(ch, HT, NT, gsemB)
        if ch + 1 < CPW:
            build_idx(ch + 1)
        for d in ga:
            d.wait()
        wa = fire_writes(cid, 0, HT)
        for d in gb:
            d.wait()
        wb = fire_writes(cid, HT, NT)
        for d in wa + wb:
            d.wait()


_MMB = 40


def _mm_body(col_ref, w_ref, b_ref, o_ref):
    acc = jnp.broadcast_to(b_ref[...], (_MMB * CH, COUT))
    for g in range(NQUAD):
        x = col_ref[:, g].reshape(_MMB * CH, 128)
        acc = acc + jnp.dot(x, w_ref[g], preferred_element_type=jnp.float32)
    o_ref[...] = acc


def kernel(inp_features, inp_positions, out_positions, voxel_size, kernel, bias):
    n_in = inp_features.shape[0]
    n_out = out_positions.shape[0]
    v = jnp.float32(voxel_size)
    ic = jnp.round(inp_positions / v).astype(jnp.int32) + 1
    oc = jnp.round(out_positions / v).astype(jnp.int32) + 1
    lin_in = (ic[:, 0] * GP + ic[:, 1]) * GP + ic[:, 2]
    lin_out = (oc[:, 0] * GP + oc[:, 1]) * GP + oc[:, 2]

    lin_in_p = jnp.full((NP,), -1, jnp.int32).at[:n_in].set(lin_in)
    feats_p = jnp.zeros((NP, CIN), jnp.float32).at[:n_in].set(inp_features)
    lin_out_p = jnp.full((NP,), GP * GP + GP + 1, jnp.int32).at[:n_out].set(lin_out)
    zeros_sh = jnp.zeros((HALF, CIN), jnp.float32)

    grid = _scatter_grid(lin_in_p, feats_p, zeros_sh)
    col = _gather_cols(grid, lin_out_p)

    w28 = jnp.concatenate(
        [kernel.reshape(NTAP, CIN, COUT),
         jnp.zeros((1, CIN, COUT), jnp.float32)], axis=0)
    wq = w28.reshape(NQUAD, 4 * CIN, COUT)
    b2 = bias.reshape(1, COUT)
    out = pl.pallas_call(
        _mm_body,
        grid=(NCHUNK // _MMB,),
        in_specs=[pl.BlockSpec((_MMB, NQUAD, CH, 128), lambda i: (i, 0, 0, 0)),
                  pl.BlockSpec((NQUAD, 4 * CIN, COUT), lambda i: (0, 0, 0)),
                  pl.BlockSpec((1, COUT), lambda i: (0, 0))],
        out_specs=pl.BlockSpec((_MMB * CH, COUT), lambda i: (i, 0)),
        out_shape=jax.ShapeDtypeStruct((NP, COUT), jnp.float32),
    )(col, wq, b2)
    return out[:n_out]


# speedup vs baseline: 3.9593x; 1.0082x over previous
import functools

import jax
import jax.numpy as jnp
from jax import lax
from jax.experimental import pallas as pl
from jax.experimental.pallas import tpu as pltpu
from jax.experimental.pallas import tpu_sc as plsc

G = 40
GP = G + 2
NCELL = GP * GP * GP
HALF = 37120
RPAD = 2 * HALF
DUMMY = HALF
SH_ROWS = HALF + 8
CIN = 32
COUT = 32
NP = 10240
PPT = 640
GPT = 320
CH = 32
NTAP = 27

OFFS = [(dx * GP + dy) * GP + dz
        for dx in (-1, 0, 1) for dy in (-1, 0, 1) for dz in (-1, 0, 1)]
OFFS28 = OFFS + [0]
NQUAD = 7

_MESH = plsc.VectorSubcoreMesh(core_axis_name="c", subcore_axis_name="s")
_SC_PARAMS = pltpu.CompilerParams(use_tc_tiling_on_sc=False)


@functools.partial(
    pl.kernel,
    mesh=_MESH,
    out_type=jax.ShapeDtypeStruct((RPAD, CIN), jnp.float32),
    scratch_types=[
        pltpu.VMEM((PPT,), jnp.int32),
        pltpu.VMEM((5, 128), jnp.int32),
        pltpu.VMEM((2, 128, CIN), jnp.float32),
        pltpu.VMEM_SHARED((SH_ROWS, CIN), jnp.float32),
        pltpu.SemaphoreType.DMA,
        pltpu.SemaphoreType.DMA,
    ],
    compiler_params=_SC_PARAMS,
)
def _scatter_grid(lin_hbm, feat_hbm, zeros_hbm, grid_hbm,
                  linbuf, idxbuf, featbuf, shared, zsem, fsem):
    c = lax.axis_index("c")
    s = lax.axis_index("s")
    zd = pltpu.async_copy(zeros_hbm.at[pl.ds(s * 2320, 2320)],
                          shared.at[pl.ds(s * 2320, 2320)], zsem)
    ld = pltpu.async_copy(lin_hbm.at[pl.ds(s * PPT, PPT)], linbuf, fsem)
    f0 = pltpu.async_copy(feat_hbm.at[pl.ds(s * PPT, 128)],
                          featbuf.at[0], fsem)
    ld.wait()
    base = c * HALF
    basev = jnp.broadcast_to(base, (16,))
    for k in range(5):
        for j in range(8):
            lv = linbuf[pl.ds(k * 128 + j * 16, 16)]
            loc = lv - basev
            ok = (loc >= 0) & (loc < HALF)
            idxbuf[k, pl.ds(j * 16, 16)] = jnp.where(ok, loc, DUMMY)
    zd.wait()
    plsc.subcore_barrier()
    fprev = f0
    for k in range(5):
        fprev.wait()
        if k + 1 < 5:
            fprev = pltpu.async_copy(
                feat_hbm.at[pl.ds(s * PPT + (k + 1) * 128, 128)],
                featbuf.at[(k + 1) % 2], fsem)
        pltpu.sync_copy(featbuf.at[k % 2], shared.at[idxbuf.at[k]], add=True)
    plsc.subcore_barrier()
    pltpu.sync_copy(shared.at[pl.ds(s * 2320, 2320)],
                    grid_hbm.at[pl.ds(base + s * 2320, 2320)])


NCHUNK = NP // CH
CPW = GPT // CH


@functools.partial(
    pl.kernel,
    mesh=_MESH,
    out_type=jax.ShapeDtypeStruct((NCHUNK, NQUAD, CH, 128), jnp.float32),
    scratch_types=[
        pltpu.VMEM((GPT,), jnp.int32),
        pltpu.VMEM((2, NQUAD, 4 * CH), jnp.int32),
        pltpu.VMEM((NQUAD, 4 * CH, CIN), jnp.float32),
        pltpu.SemaphoreType.DMA,
        pltpu.SemaphoreType.DMA,
        pltpu.SemaphoreType.DMA,
    ],
    compiler_params=_SC_PARAMS,
)
def _gather_cols(grid_hbm, lin_hbm, col_hbm, linbuf, idxbuf, gbuf,
                 gsemA, gsemB, wsem):
    c = lax.axis_index("c")
    s = lax.axis_index("s")
    w = s * 2 + c
    base = w * GPT
    pltpu.sync_copy(lin_hbm.at[pl.ds(base, GPT)], linbuf)

    def build_idx(ch):
        ib = idxbuf.at[ch % 2]
        for g in range(NQUAD):
            for v in range(8):
                pv = linbuf[pl.ds(ch * CH + 16 * (v % 2), 16)]
                ib[g, pl.ds(v * 16, 16)] = pv + OFFS28[4 * g + v // 2]

    def fire_gathers(ch, lo, hi, sem):
        ib = idxbuf.at[ch % 2]
        return [pltpu.async_copy(grid_hbm.at[ib.at[g]], gbuf.at[g], sem)
                for g in range(lo, hi)]

    def fire_writes(cid, lo, hi):
        return [pltpu.async_copy(
            gbuf.at[g, pl.ds(q * CH, CH)],
            col_hbm.at[cid, g, :, pl.ds(q * CIN, CIN)],
            wsem) for g in range(lo, hi) for q in range(4)]

    build_idx(0)
    for ch in range(CPW):
        cid = w * CPW + ch
        ga = fire_gathers(ch, 0, 4, gsemA)
        gb = fire_gathers(ch, 4, NQUAD, gsemB)
        if ch + 1 < CPW:
            build_idx(ch + 1)
        for d in ga:
            d.wait()
        wa = fire_writes(cid, 0, 4)
        for d in gb:
            d.wait()
        wb = fire_writes(cid, 4, NQUAD)
        for d in wa + wb:
            d.wait()


_MMB = 40


def _mm_body(col_ref, w_ref, b_ref, o_ref):
    acc = jnp.broadcast_to(b_ref[...], (_MMB * CH, COUT))
    for g in range(NQUAD):
        x = col_ref[:, g].reshape(_MMB * CH, 128)
        acc = acc + jnp.dot(x, w_ref[g], preferred_element_type=jnp.float32)
    o_ref[...] = acc


def kernel(inp_features, inp_positions, out_positions, voxel_size, kernel, bias):
    n_in = inp_features.shape[0]
    n_out = out_positions.shape[0]
    v = jnp.float32(voxel_size)
    ic = jnp.round(inp_positions / v).astype(jnp.int32) + 1
    oc = jnp.round(out_positions / v).astype(jnp.int32) + 1
    lin_in = (ic[:, 0] * GP + ic[:, 1]) * GP + ic[:, 2]
    lin_out = (oc[:, 0] * GP + oc[:, 1]) * GP + oc[:, 2]

    lin_in_p = jnp.full((NP,), -1, jnp.int32).at[:n_in].set(lin_in)
    feats_p = jnp.zeros((NP, CIN), jnp.float32).at[:n_in].set(inp_features)
    lin_out_p = jnp.full((NP,), GP * GP + GP + 1, jnp.int32).at[:n_out].set(lin_out)
    zeros_sh = jnp.zeros((HALF, CIN), jnp.float32)

    grid = _scatter_grid(lin_in_p, feats_p, zeros_sh)
    col = _gather_cols(grid, lin_out_p)

    w28 = jnp.concatenate(
        [kernel.reshape(NTAP, CIN, COUT),
         jnp.zeros((1, CIN, COUT), jnp.float32)], axis=0)
    wq = w28.reshape(NQUAD, 4 * CIN, COUT)
    b2 = bias.reshape(1, COUT)
    out = pl.pallas_call(
        _mm_body,
        grid=(NCHUNK // _MMB,),
        in_specs=[pl.BlockSpec((_MMB, NQUAD, CH, 128), lambda i: (i, 0, 0, 0)),
                  pl.BlockSpec((NQUAD, 4 * CIN, COUT), lambda i: (0, 0, 0)),
                  pl.BlockSpec((1, COUT), lambda i: (0, 0))],
        out_specs=pl.BlockSpec((_MMB * CH, COUT), lambda i: (i, 0)),
        out_shape=jax.ShapeDtypeStruct((NP, COUT), jnp.float32),
    )(col, wq, b2)
    return out[:n_out]


# speedup vs baseline: 4.1544x; 1.0493x over previous
import functools

import jax
import jax.numpy as jnp
from jax import lax
from jax.experimental import pallas as pl
from jax.experimental.pallas import tpu as pltpu
from jax.experimental.pallas import tpu_sc as plsc

G = 40
GP = G + 2
NCELL = GP * GP * GP
HALF = 37120
RPAD = 2 * HALF
DUMMY = HALF
SH_ROWS = HALF + 8
CIN = 32
COUT = 32
NP = 10240
PPT = 640
GPT = 320
CH = 32
NTAP = 27

OFFS = [(dx * GP + dy) * GP + dz
        for dx in (-1, 0, 1) for dy in (-1, 0, 1) for dz in (-1, 0, 1)]
OFFS28 = OFFS + [0]
NQUAD = 7

_MESH = plsc.VectorSubcoreMesh(core_axis_name="c", subcore_axis_name="s")
_SC_PARAMS = pltpu.CompilerParams(use_tc_tiling_on_sc=False)


@functools.partial(
    pl.kernel,
    mesh=_MESH,
    out_type=jax.ShapeDtypeStruct((RPAD, CIN), jnp.float32),
    scratch_types=[
        pltpu.VMEM((PPT,), jnp.int32),
        pltpu.VMEM((5, 128), jnp.int32),
        pltpu.VMEM((2, 128, CIN), jnp.float32),
        pltpu.VMEM_SHARED((SH_ROWS, CIN), jnp.float32),
        pltpu.SemaphoreType.DMA,
        pltpu.SemaphoreType.DMA,
    ],
    compiler_params=_SC_PARAMS,
)
def _scatter_grid(lin_hbm, feat_hbm, zeros_hbm, grid_hbm,
                  linbuf, idxbuf, featbuf, shared, zsem, fsem):
    c = lax.axis_index("c")
    s = lax.axis_index("s")
    zd = pltpu.async_copy(zeros_hbm,
                          shared.at[pl.ds(s * 2320, 2320)], zsem)
    ld = pltpu.async_copy(lin_hbm.at[pl.ds(s * PPT, PPT)], linbuf, fsem)
    f0 = pltpu.async_copy(feat_hbm.at[pl.ds(s * PPT, 128)],
                          featbuf.at[0], fsem)
    ld.wait()
    base = c * HALF
    basev = jnp.broadcast_to(base, (16,))
    for k in range(5):
        for j in range(8):
            lv = linbuf[pl.ds(k * 128 + j * 16, 16)]
            loc = lv - basev
            ok = (loc >= 0) & (loc < HALF)
            idxbuf[k, pl.ds(j * 16, 16)] = jnp.where(ok, loc, DUMMY)
    zd.wait()
    plsc.subcore_barrier()
    fprev = f0
    for k in range(5):
        fprev.wait()
        if k + 1 < 5:
            fprev = pltpu.async_copy(
                feat_hbm.at[pl.ds(s * PPT + (k + 1) * 128, 128)],
                featbuf.at[(k + 1) % 2], fsem)
        pltpu.sync_copy(featbuf.at[k % 2], shared.at[idxbuf.at[k]], add=True)
    plsc.subcore_barrier()
    pltpu.sync_copy(shared.at[pl.ds(s * 2320, 2320)],
                    grid_hbm.at[pl.ds(base + s * 2320, 2320)])


NCHUNK = NP // CH
CPW = GPT // CH


@functools.partial(
    pl.kernel,
    mesh=_MESH,
    out_type=jax.ShapeDtypeStruct((NCHUNK, NQUAD, CH, 128), jnp.float32),
    scratch_types=[
        pltpu.VMEM((GPT,), jnp.int32),
        pltpu.VMEM((2, NQUAD, 4 * CH), jnp.int32),
        pltpu.VMEM((NQUAD, 4 * CH, CIN), jnp.float32),
        pltpu.SemaphoreType.DMA,
        pltpu.SemaphoreType.DMA,
        pltpu.SemaphoreType.DMA,
    ],
    compiler_params=_SC_PARAMS,
)
def _gather_cols(grid_hbm, lin_hbm, col_hbm, linbuf, idxbuf, gbuf,
                 gsemA, gsemB, wsem):
    c = lax.axis_index("c")
    s = lax.axis_index("s")
    w = s * 2 + c
    base = w * GPT
    pltpu.sync_copy(lin_hbm.at[pl.ds(base, GPT)], linbuf)

    def build_idx(ch):
        ib = idxbuf.at[ch % 2]
        for g in range(NQUAD):
            for v in range(8):
                pv = linbuf[pl.ds(ch * CH + 16 * (v % 2), 16)]
                ib[g, pl.ds(v * 16, 16)] = pv + OFFS28[4 * g + v // 2]

    def fire_gathers(ch, lo, hi, sem):
        ib = idxbuf.at[ch % 2]
        return [pltpu.async_copy(grid_hbm.at[ib.at[g]], gbuf.at[g], sem)
                for g in range(lo, hi)]

    def fire_writes(cid, lo, hi):
        return [pltpu.async_copy(
            gbuf.at[g, pl.ds(q * CH, CH)],
            col_hbm.at[cid, g, :, pl.ds(q * CIN, CIN)],
            wsem) for g in range(lo, hi) for q in range(4)]

    build_idx(0)
    for ch in range(CPW):
        cid = w * CPW + ch
        ga = fire_gathers(ch, 0, 4, gsemA)
        gb = fire_gathers(ch, 4, NQUAD, gsemB)
        if ch + 1 < CPW:
            build_idx(ch + 1)
        for d in ga:
            d.wait()
        wa = fire_writes(cid, 0, 4)
        for d in gb:
            d.wait()
        wb = fire_writes(cid, 4, NQUAD)
        for d in wa + wb:
            d.wait()


_MMB = 40


def _mm_body(col_ref, w_ref, b_ref, o_ref):
    acc = jnp.broadcast_to(b_ref[...], (_MMB * CH, COUT))
    for g in range(NQUAD):
        x = col_ref[:, g].reshape(_MMB * CH, 128)
        acc = acc + jnp.dot(x, w_ref[g], preferred_element_type=jnp.float32)
    o_ref[...] = acc


def kernel(inp_features, inp_positions, out_positions, voxel_size, kernel, bias):
    n_in = inp_features.shape[0]
    n_out = out_positions.shape[0]
    v = jnp.float32(voxel_size)
    ic = jnp.round(inp_positions / v).astype(jnp.int32) + 1
    oc = jnp.round(out_positions / v).astype(jnp.int32) + 1
    lin_in = (ic[:, 0] * GP + ic[:, 1]) * GP + ic[:, 2]
    lin_out = (oc[:, 0] * GP + oc[:, 1]) * GP + oc[:, 2]

    lin_in_p = jnp.full((NP,), -1, jnp.int32).at[:n_in].set(lin_in)
    feats_p = jnp.zeros((NP, CIN), jnp.float32).at[:n_in].set(inp_features)
    lin_out_p = jnp.full((NP,), GP * GP + GP + 1, jnp.int32).at[:n_out].set(lin_out)
    zeros_sh = jnp.zeros((2320, CIN), jnp.float32)

    grid = _scatter_grid(lin_in_p, feats_p, zeros_sh)
    col = _gather_cols(grid, lin_out_p)

    w28 = jnp.concatenate(
        [kernel.reshape(NTAP, CIN, COUT),
         jnp.zeros((1, CIN, COUT), jnp.float32)], axis=0)
    wq = w28.reshape(NQUAD, 4 * CIN, COUT)
    b2 = bias.reshape(1, COUT)
    out = pl.pallas_call(
        _mm_body,
        grid=(NCHUNK // _MMB,),
        in_specs=[pl.BlockSpec((_MMB, NQUAD, CH, 128), lambda i: (i, 0, 0, 0)),
                  pl.BlockSpec((NQUAD, 4 * CIN, COUT), lambda i: (0, 0, 0)),
                  pl.BlockSpec((1, COUT), lambda i: (0, 0))],
        out_specs=pl.BlockSpec((_MMB * CH, COUT), lambda i: (i, 0)),
        out_shape=jax.ShapeDtypeStruct((NP, COUT), jnp.float32),
    )(col, wq, b2)
    return out[:n_out]
